# baseline (speedup 1.0000x reference)
import functools

import jax
import jax.numpy as jnp
from jax.experimental import pallas as pl
from jax.experimental.pallas import tpu as pltpu


def _knn_body(nq, n_steps, bn, ch, q_ref, mb_ref,
              scores_ref, img_ref, idx_ref,
              acc_scr, msq_scr, vmin_ref, vidx_ref):
    step = pl.program_id(0)
    qp = q_ref.shape[0]

    @pl.when(step == 0)
    def _init():
        vmin_ref[...] = jnp.full(vmin_ref.shape, jnp.inf, jnp.float32)
        vidx_ref[...] = jnp.zeros(vidx_ref.shape, jnp.float32)

    q = q_ref[...]
    mb = mb_ref[...]
    msq_scr[...] = jnp.sum(mb * mb, axis=1, keepdims=True)
    acc_scr[...] = jax.lax.dot_general(
        mb, q * -2.0, (((1,), (1,)), ((), ())),
        preferred_element_type=jnp.float32)
    qsq_row = jnp.sum(q * q, axis=1)[None, :]

    nch = bn // ch

    def body(r, carry):
        bv, bi = carry
        a = acc_scr[pl.ds(r * ch, ch), :]
        ms = msq_scr[pl.ds(r * ch, ch), :]
        d2 = (qsq_row + a) + ms
        better = d2 < bv
        bv = jnp.where(better, d2, bv)
        bi = jnp.where(better, r.astype(jnp.float32), bi)
        return bv, bi

    bv0 = jnp.full((ch, qp), jnp.inf, jnp.float32)
    bi0 = jnp.zeros((ch, qp), jnp.float32)
    bv, bi = jax.lax.fori_loop(0, nch, body, (bv0, bi0))

    s_io = jax.lax.broadcasted_iota(jnp.int32, (ch, qp), 0).astype(jnp.float32)
    rows = bi * jnp.float32(ch) + s_io
    bmin = jnp.min(bv, axis=0, keepdims=True)
    widx = jnp.min(jnp.where(bv == bmin, rows, jnp.float32(bn)),
                   axis=0, keepdims=True)

    better = bmin < vmin_ref[...]
    base = (step * bn).astype(jnp.float32)
    vidx_ref[...] = jnp.where(better, widx + base, vidx_ref[...])
    vmin_ref[...] = jnp.where(better, bmin, vmin_ref[...])

    @pl.when(step == n_steps - 1)
    def _finish():
        s = jnp.sqrt(jnp.maximum(vmin_ref[...], 0.0) + 1e-12)
        scores_ref[...] = s
        lane_q = jax.lax.broadcasted_iota(jnp.int32, (1, qp), 1)
        img_ref[...] = jnp.max(
            jnp.where(lane_q < nq, s, -jnp.inf)).reshape(1, 1)
        idx_ref[...] = vidx_ref[...].astype(jnp.int32)


def kernel(queries, memory_bank, k):
    Q, D = queries.shape
    N = memory_bank.shape[0]
    BN = 4000
    CH = 16
    n_steps = N // BN
    QP = -(-Q // 128) * 128

    qp_arr = jnp.pad(queries, ((0, QP - Q), (0, 0)))

    scores, img, idx = pl.pallas_call(
        functools.partial(_knn_body, Q, n_steps, BN, CH),
        grid=(n_steps,),
        in_specs=[
            pl.BlockSpec((QP, D), lambda i: (0, 0)),
            pl.BlockSpec((BN, D), lambda i: (i, 0)),
        ],
        out_specs=[
            pl.BlockSpec((1, QP), lambda i: (0, 0)),
            pl.BlockSpec((1, 1), lambda i: (0, 0)),
            pl.BlockSpec((1, QP), lambda i: (0, 0)),
        ],
        out_shape=[
            jax.ShapeDtypeStruct((1, QP), jnp.float32),
            jax.ShapeDtypeStruct((1, 1), jnp.float32),
            jax.ShapeDtypeStruct((1, QP), jnp.int32),
        ],
        scratch_shapes=[
            pltpu.VMEM((BN, QP), jnp.float32),
            pltpu.VMEM((BN, 1), jnp.float32),
            pltpu.VMEM((1, QP), jnp.float32),
            pltpu.VMEM((1, QP), jnp.float32),
        ],
    )(qp_arr, memory_bank)

    kf = jnp.asarray(k, jnp.float32)
    patch_scores = scores[0, :Q] / kf
    image_score = img[0, 0] / kf
    nn_idx = idx[0, :Q].reshape(Q, 1)
    return (patch_scores, image_score, nn_idx)


# speedup vs baseline: 275.6899x; 275.6899x over previous
import functools

import jax
import jax.numpy as jnp
from jax.experimental import pallas as pl
from jax.experimental.pallas import tpu as pltpu


def _knn_body(n_steps, bn, q_ref, mb_ref, lane_ref,
              scores_ref, img_ref, idx_ref, vmin_ref, vidx_ref):
    step = pl.program_id(0)

    @pl.when(step == 0)
    def _init():
        vmin_ref[...] = jnp.full(vmin_ref.shape, jnp.inf, jnp.float32)
        vidx_ref[...] = jnp.zeros(vidx_ref.shape, jnp.int32)

    q = q_ref[...]
    mb = mb_ref[...]
    acc = jax.lax.dot_general(
        q * -2.0, mb, (((1,), (1,)), ((), ())),
        preferred_element_type=jnp.float32)
    q_sq = jnp.sum(q * q, axis=1, keepdims=True)
    m_sq = jnp.sum(mb * mb, axis=1)
    d2 = (q_sq + acc) + m_sq[None, :]

    bmin = jnp.min(d2, axis=1, keepdims=True)
    bidx_f = jnp.min(jnp.where(d2 == bmin, lane_ref[...], jnp.float32(bn)),
                     axis=1, keepdims=True)
    bidx = bidx_f.astype(jnp.int32) + step * bn

    better = bmin < vmin_ref[...]
    vidx_ref[...] = jnp.where(better, bidx, vidx_ref[...])
    vmin_ref[...] = jnp.where(better, bmin, vmin_ref[...])

    @pl.when(step == n_steps - 1)
    def _finish():
        s = jnp.sqrt(jnp.maximum(vmin_ref[...], 0.0) + 1e-12)
        scores_ref[...] = s
        img_ref[...] = jnp.max(s).reshape(1, 1)
        idx_ref[...] = vidx_ref[...]


def kernel(queries, memory_bank, k):
    Q, D = queries.shape
    N = memory_bank.shape[0]
    BN = 2000
    n_steps = N // BN

    scores, img, idx = pl.pallas_call(
        functools.partial(_knn_body, n_steps, BN),
        grid=(n_steps,),
        in_specs=[
            pl.BlockSpec((Q, D), lambda i: (0, 0)),
            pl.BlockSpec((BN, D), lambda i: (i, 0)),
            pl.BlockSpec((1, BN), lambda i: (0, 0)),
        ],
        out_specs=[
            pl.BlockSpec((Q, 1), lambda i: (0, 0)),
            pl.BlockSpec((1, 1), lambda i: (0, 0)),
            pl.BlockSpec((Q, 1), lambda i: (0, 0)),
        ],
        out_shape=[
            jax.ShapeDtypeStruct((Q, 1), jnp.float32),
            jax.ShapeDtypeStruct((1, 1), jnp.float32),
            jax.ShapeDtypeStruct((Q, 1), jnp.int32),
        ],
        scratch_shapes=[
            pltpu.VMEM((Q, 1), jnp.float32),
            pltpu.VMEM((Q, 1), jnp.int32),
        ],
    )(queries, memory_bank, jnp.arange(BN, dtype=jnp.float32)[None, :])

    kf = jnp.asarray(k, jnp.float32)
    patch_scores = scores[:, 0] / kf
    image_score = img[0, 0] / kf
    return (patch_scores, image_score, idx)


# speedup vs baseline: 298.8096x; 1.0839x over previous
import functools

import jax
import jax.numpy as jnp
from jax.experimental import pallas as pl
from jax.experimental.pallas import tpu as pltpu


def _knn_body(n_steps, bn, q_ref, mb_ref, lane_ref,
              scores_ref, img_ref, idx_ref, vmin_ref, vidx_ref):
    step = pl.program_id(0)

    @pl.when(step == 0)
    def _init():
        vmin_ref[...] = jnp.full(vmin_ref.shape, jnp.inf, jnp.float32)
        vidx_ref[...] = jnp.zeros(vidx_ref.shape, jnp.int32)

    q = q_ref[...]
    mb = mb_ref[...]
    acc = jax.lax.dot_general(
        q * -2.0, mb, (((1,), (1,)), ((), ())),
        preferred_element_type=jnp.float32)
    q_sq = jnp.sum(q * q, axis=1, keepdims=True)
    m_sq = jnp.sum(mb * mb, axis=1)
    d2 = (q_sq + acc) + m_sq[None, :]

    bmin = jnp.min(d2, axis=1, keepdims=True)
    bidx_f = jnp.min(jnp.where(d2 == bmin, lane_ref[...], jnp.float32(bn)),
                     axis=1, keepdims=True)
    bidx = bidx_f.astype(jnp.int32) + step * bn

    better = bmin < vmin_ref[...]
    vidx_ref[...] = jnp.where(better, bidx, vidx_ref[...])
    vmin_ref[...] = jnp.where(better, bmin, vmin_ref[...])

    @pl.when(step == n_steps - 1)
    def _finish():
        s = jnp.sqrt(jnp.maximum(vmin_ref[...], 0.0) + 1e-12)
        scores_ref[...] = s
        img_ref[...] = jnp.max(s).reshape(1, 1)
        idx_ref[...] = vidx_ref[...]


def kernel(queries, memory_bank, k):
    Q, D = queries.shape
    N = memory_bank.shape[0]
    BN = 5000
    n_steps = N // BN

    scores, img, idx = pl.pallas_call(
        functools.partial(_knn_body, n_steps, BN),
        grid=(n_steps,),
        in_specs=[
            pl.BlockSpec((Q, D), lambda i: (0, 0)),
            pl.BlockSpec((BN, D), lambda i: (i, 0)),
            pl.BlockSpec((1, BN), lambda i: (0, 0)),
        ],
        out_specs=[
            pl.BlockSpec((Q, 1), lambda i: (0, 0)),
            pl.BlockSpec((1, 1), lambda i: (0, 0)),
            pl.BlockSpec((Q, 1), lambda i: (0, 0)),
        ],
        out_shape=[
            jax.ShapeDtypeStruct((Q, 1), jnp.float32),
            jax.ShapeDtypeStruct((1, 1), jnp.float32),
            jax.ShapeDtypeStruct((Q, 1), jnp.int32),
        ],
        scratch_shapes=[
            pltpu.VMEM((Q, 1), jnp.float32),
            pltpu.VMEM((Q, 1), jnp.int32),
        ],
    )(queries, memory_bank, jnp.arange(BN, dtype=jnp.float32)[None, :])

    kf = jnp.asarray(k, jnp.float32)
    patch_scores = scores[:, 0] / kf
    image_score = img[0, 0] / kf
    return (patch_scores, image_score, idx)
